# baseline (speedup 1.0000x reference)
import jax
import jax.numpy as jnp
from jax.experimental import pallas as pl
from jax.experimental.pallas import tpu as pltpu

_PATCH = 2
_EMBED = 768
_MAX_TOK = 2048
_NBUF = 4


def _body(x_ref, w_ref, bpos_ref, o_ref, buf, sems):
    i = pl.program_id(0)
    n = pl.num_programs(0)
    T = bpos_ref.shape[0]
    g = i // 8
    f = (i // 2) % 4
    v = i % 2
    b = i % _NBUF

    def _copy(bb, dst_g, dst_f, dst_v):
        return pltpu.make_async_copy(
            buf.at[bb],
            o_ref.at[dst_g, dst_f, pl.ds(dst_v * T, T), :],
            sems.at[bb],
        )

    @pl.when(i >= _NBUF)
    def _():
        _copy(b, g, f, v).wait()

    acc = jnp.dot(x_ref[0, 0], w_ref[...], preferred_element_type=jnp.float32)
    buf[b] = acc + bpos_ref[...]
    _copy(b, g, f, v).start()

    @pl.when(i == n - 1)
    def _():
        for off in range(_NBUF):
            _copy((b - off) % _NBUF, g, f, v).wait()


def kernel(latent, Wp, bp, pos_embed):
    B, C, F, H, W = latent.shape
    ph, pw = H // _PATCH, W // _PATCH
    T = ph * pw
    gsz = _MAX_TOK // T
    ng = B // gsz
    K = C * _PATCH * _PATCH

    x = latent.reshape(B, C, F, ph, _PATCH, pw, _PATCH)
    x = x.transpose(0, 2, 3, 5, 1, 4, 6).reshape(B, F, T, K)
    w = Wp.reshape(_EMBED, K).T
    bpos = pos_embed + bp[None, :]

    grid = (ng * F * gsz,)
    out = pl.pallas_call(
        _body,
        grid=grid,
        in_specs=[
            pl.BlockSpec(
                (1, 1, T, K),
                lambda i: (gsz * (i // (F * gsz)) + i % gsz, (i // gsz) % F, 0, 0),
            ),
            pl.BlockSpec((K, _EMBED), lambda i: (0, 0)),
            pl.BlockSpec((T, _EMBED), lambda i: (0, 0)),
        ],
        out_specs=pl.BlockSpec(memory_space=pl.ANY),
        out_shape=jax.ShapeDtypeStruct((ng, F, _MAX_TOK, _EMBED), jnp.float32),
        scratch_shapes=[
            pltpu.VMEM((_NBUF, T, _EMBED), jnp.float32),
            pltpu.SemaphoreType.DMA((_NBUF,)),
        ],
        compiler_params=pltpu.CompilerParams(
            dimension_semantics=("arbitrary",),
        ),
    )(x, w, bpos)

    batched_idx = jnp.tile(
        jnp.repeat(jnp.arange(gsz, dtype=jnp.int32), T), (ng, 1)
    )
    return (out, batched_idx)
